# Initial kernel scaffold; baseline (speedup 1.0000x reference)
import jax
import jax.numpy as jnp
from jax.experimental import pallas as pl


def kernel(x, edge_index_rel0, edge_index_rel1, self_idx, W0_self, W0_r0, W0_r1, b0, W1_self, W1_r0, W1_r1, b1):
    raise NotImplementedError("write your pallas kernel here")


# speedup vs baseline: 2.8459x; 2.8459x over previous
import functools

import jax
import jax.numpy as jnp
from jax import lax
from jax.experimental import pallas as pl
from jax.experimental.pallas import tpu as pltpu
from jax.experimental.pallas import tpu_sc as plsc

N = 10000
D = 128
E = 160000
EPS0 = 0.1
EPS1 = 0.1

NC = 2
NS = 16
NW = NC * NS
B = 128
NB = 40
E_PAD = NW * NB * B
CH = 632
ACC_ROWS = CH * NS
ZROWS = 128
CH_LAST = N - CH * (NS - 1)


def _sc_body(h_hbm, src0_hbm, dst0_hbm, src1_hbm, dst1_hbm, out_hbm,
             src_idx, dst_idx, rows0, rows1, sem0, sem1, acc):
    c = lax.axis_index("c")
    s = lax.axis_index("s")
    wid = c * NS + s

    zeros16 = jnp.zeros((16,), jnp.float32)

    def _zrow(i, carry):
        for k in range(8):
            rows0[i, pl.ds(k * 16, 16)] = zeros16
        return carry

    def _zero_acc():
        lax.fori_loop(0, ZROWS, _zrow, 0)
        zbase = s * CH
        nfull = CH // ZROWS
        for k in range(nfull):
            pltpu.sync_copy(rows0, acc.at[pl.ds(zbase + k * ZROWS, ZROWS)])
        rem = CH - nfull * ZROWS
        if rem:
            pltpu.sync_copy(rows0.at[pl.ds(0, rem)],
                            acc.at[pl.ds(zbase + nfull * ZROWS, rem)])

    _zero_acc()
    plsc.subcore_barrier()

    for r, (src_hbm, dst_hbm) in enumerate(
            ((src0_hbm, dst0_hbm), (src1_hbm, dst1_hbm))):
        pltpu.sync_copy(src_hbm.at[pl.ds(wid * NB, NB)], src_idx)
        pltpu.sync_copy(dst_hbm.at[pl.ds(wid * NB, NB)], dst_idx)

        pltpu.async_copy(h_hbm.at[src_idx.at[0]], rows0, sem0)

        def _pair(j, carry):
            pltpu.async_copy(h_hbm.at[src_idx.at[2 * j + 1]], rows1, sem1)
            pltpu.make_async_copy(h_hbm.at[src_idx.at[2 * j]], rows0,
                                  sem0).wait()
            pltpu.sync_copy(rows0, acc.at[dst_idx.at[2 * j]], add=True)

            @pl.when(j < NB // 2 - 1)
            def _():
                pltpu.async_copy(h_hbm.at[src_idx.at[2 * j + 2]], rows0, sem0)

            pltpu.make_async_copy(h_hbm.at[src_idx.at[2 * j + 1]], rows1,
                                  sem1).wait()
            pltpu.sync_copy(rows1, acc.at[dst_idx.at[2 * j + 1]], add=True)
            return carry

        lax.fori_loop(0, NB // 2, _pair, 0)

        plsc.subcore_barrier()
        obase = s * CH
        dst_row = (r * NC + c) * N + obase

        @pl.when(s < NS - 1)
        def _():
            pltpu.sync_copy(acc.at[pl.ds(obase, CH)],
                            out_hbm.at[pl.ds(dst_row, CH)])

        @pl.when(s == NS - 1)
        def _():
            pltpu.sync_copy(acc.at[pl.ds(obase, CH_LAST)],
                            out_hbm.at[pl.ds(dst_row, CH_LAST)])
        if r == 0:
            plsc.subcore_barrier()
            _zero_acc()
            plsc.subcore_barrier()


_sc_segsum = pl.kernel(
    _sc_body,
    out_type=jax.ShapeDtypeStruct((4 * N, D), jnp.float32),
    mesh=plsc.VectorSubcoreMesh(core_axis_name="c", subcore_axis_name="s",
                                num_cores=NC, num_subcores=NS),
    scratch_types=[
        pltpu.VMEM((NB, B), jnp.int32),
        pltpu.VMEM((NB, B), jnp.int32),
        pltpu.VMEM((B, D), jnp.float32),
        pltpu.VMEM((B, D), jnp.float32),
        pltpu.SemaphoreType.DMA,
        pltpu.SemaphoreType.DMA,
        pltpu.VMEM_SHARED((ACC_ROWS, D), jnp.float32),
    ],
)


def _fuse_body(eps, x_ref, p_ref, ws_ref, w0_ref, w1_ref, b_ref, o_ref):
    a = jnp.dot(x_ref[...] * (1.0 + eps), ws_ref[...],
                preferred_element_type=jnp.float32)
    a = a + jnp.dot(p_ref[0] + p_ref[1], w0_ref[...],
                    preferred_element_type=jnp.float32)
    a = a + jnp.dot(p_ref[2] + p_ref[3], w1_ref[...],
                    preferred_element_type=jnp.float32)
    o_ref[...] = jnp.maximum(a + b_ref[...], 0.0)


def _tc_fuse(x, p, w_self, w_r0, w_r1, b, eps):
    R = 1000
    grid = N // R
    return pl.pallas_call(
        functools.partial(_fuse_body, eps),
        grid=(grid,),
        in_specs=[
            pl.BlockSpec((R, D), lambda i: (i, 0)),
            pl.BlockSpec((4, R, D), lambda i: (0, i, 0)),
            pl.BlockSpec((D, D), lambda i: (0, 0)),
            pl.BlockSpec((D, D), lambda i: (0, 0)),
            pl.BlockSpec((D, D), lambda i: (0, 0)),
            pl.BlockSpec((1, D), lambda i: (0, 0)),
        ],
        out_specs=pl.BlockSpec((R, D), lambda i: (i, 0)),
        out_shape=jax.ShapeDtypeStruct((N, D), jnp.float32),
    )(x, p, w_self, w_r0, w_r1, b)


def _pad_edges(ei):
    src = jnp.concatenate(
        [ei[0], jnp.zeros((E_PAD - E,), jnp.int32)]).reshape(E_PAD // B, B)
    dst = jnp.concatenate(
        [ei[1], jnp.full((E_PAD - E,), N, jnp.int32)]).reshape(E_PAD // B, B)
    return src, dst


def kernel(x, edge_index_rel0, edge_index_rel1, self_idx,
           W0_self, W0_r0, W0_r1, b0, W1_self, W1_r0, W1_r1, b1):
    del self_idx
    src0, dst0 = _pad_edges(edge_index_rel0)
    src1, dst1 = _pad_edges(edge_index_rel1)

    p = _sc_segsum(x, src0, dst0, src1, dst1).reshape(4, N, D)
    h = _tc_fuse(x, p, W0_self, W0_r0, W0_r1, b0.reshape(1, D), EPS0)

    p = _sc_segsum(h, src0, dst0, src1, dst1).reshape(4, N, D)
    out = _tc_fuse(h, p, W1_self, W1_r0, W1_r1, b1.reshape(1, D), EPS1)
    return out


# speedup vs baseline: 3.8334x; 1.3470x over previous
import functools

import jax
import jax.numpy as jnp
from jax import lax
from jax.experimental import pallas as pl
from jax.experimental.pallas import tpu as pltpu
from jax.experimental.pallas import tpu_sc as plsc

N = 10000
D = 128
E = 160000
EPS0 = 0.1
EPS1 = 0.1

NC = 2
NS = 16
NW = NC * NS
B = 128
NB = 40
E_PAD = NW * NB * B
CH = 632
ACC_ROWS = CH * NS
ZROWS = 128
CH_LAST = N - CH * (NS - 1)


def _sc_body(h_hbm, src0_hbm, dst0_hbm, src1_hbm, dst1_hbm, z_hbm, out_hbm,
             src_idx, dst_idx, rows0, rows1, fbuf, sem0, sem1, ssem, acc):
    c = lax.axis_index("c")
    s = lax.axis_index("s")
    wid = c * NS + s

    def _zero_acc():
        pltpu.sync_copy(z_hbm, acc.at[pl.ds(s * CH, CH)])

    _zero_acc()
    plsc.subcore_barrier()

    mask_hi = jnp.full((16,), -65536, jnp.int32)

    def _upconvert(rows_w):
        def _crow(i, carry):
            for g in range(4):
                w = rows_w[i, pl.ds(16 * g, 16)]
                a = plsc.bitcast(w << 16, jnp.float32)
                b = plsc.bitcast(w & mask_hi, jnp.float32)
                fbuf[i, pl.ds(32 * g, 16)] = a
                fbuf[i, pl.ds(32 * g + 16, 16)] = b
            return carry

        lax.fori_loop(0, B, _crow, 0)

    for r, (src_hbm, dst_hbm) in enumerate(
            ((src0_hbm, dst0_hbm), (src1_hbm, dst1_hbm))):
        pltpu.sync_copy(src_hbm.at[pl.ds(wid * NB, NB)], src_idx)
        pltpu.sync_copy(dst_hbm.at[pl.ds(wid * NB, NB)], dst_idx)

        pltpu.async_copy(h_hbm.at[src_idx.at[0]], rows0, sem0)

        def _pair(j, carry):
            pltpu.async_copy(h_hbm.at[src_idx.at[2 * j + 1]], rows1, sem1)
            pltpu.make_async_copy(h_hbm.at[src_idx.at[2 * j]], rows0,
                                  sem0).wait()

            @pl.when(j > 0)
            def _():
                pltpu.make_async_copy(fbuf, acc.at[dst_idx.at[0]],
                                      ssem).wait()

            _upconvert(rows0)
            pltpu.async_copy(fbuf, acc.at[dst_idx.at[2 * j]], ssem, add=True)

            @pl.when(j < NB // 2 - 1)
            def _():
                pltpu.async_copy(h_hbm.at[src_idx.at[2 * j + 2]], rows0, sem0)

            pltpu.make_async_copy(h_hbm.at[src_idx.at[2 * j + 1]], rows1,
                                  sem1).wait()
            pltpu.make_async_copy(fbuf, acc.at[dst_idx.at[0]], ssem).wait()
            _upconvert(rows1)
            pltpu.async_copy(fbuf, acc.at[dst_idx.at[2 * j + 1]], ssem,
                             add=True)
            return carry

        lax.fori_loop(0, NB // 2, _pair, 0)
        pltpu.make_async_copy(fbuf, acc.at[dst_idx.at[0]], ssem).wait()

        plsc.subcore_barrier()
        obase = s * CH
        dst_row = (r * NC + c) * N + obase

        @pl.when(s < NS - 1)
        def _():
            pltpu.sync_copy(acc.at[pl.ds(obase, CH)],
                            out_hbm.at[pl.ds(dst_row, CH)])

        @pl.when(s == NS - 1)
        def _():
            pltpu.sync_copy(acc.at[pl.ds(obase, CH_LAST)],
                            out_hbm.at[pl.ds(dst_row, CH_LAST)])
        if r == 0:
            plsc.subcore_barrier()
            _zero_acc()
            plsc.subcore_barrier()


_sc_segsum = pl.kernel(
    _sc_body,
    out_type=jax.ShapeDtypeStruct((4 * N, D), jnp.float32),
    mesh=plsc.VectorSubcoreMesh(core_axis_name="c", subcore_axis_name="s",
                                num_cores=NC, num_subcores=NS),
    compiler_params=pltpu.CompilerParams(use_tc_tiling_on_sc=False,
                                         needs_layout_passes=False),
    scratch_types=[
        pltpu.VMEM((NB, B), jnp.int32),
        pltpu.VMEM((NB, B), jnp.int32),
        pltpu.VMEM((B, D // 2), jnp.int32),
        pltpu.VMEM((B, D // 2), jnp.int32),
        pltpu.VMEM((B, D), jnp.float32),
        pltpu.SemaphoreType.DMA,
        pltpu.SemaphoreType.DMA,
        pltpu.SemaphoreType.DMA,
        pltpu.VMEM_SHARED((ACC_ROWS, D), jnp.float32),
    ],
)


def _fuse_body(eps, x_ref, p_ref, ws_ref, w0_ref, w1_ref, b_ref, o_ref):
    a = jnp.dot(x_ref[...] * (1.0 + eps), ws_ref[...],
                preferred_element_type=jnp.float32)
    p0 = p_ref[0].astype(jnp.float32) + p_ref[1].astype(jnp.float32)
    p1 = p_ref[2].astype(jnp.float32) + p_ref[3].astype(jnp.float32)
    a = a + jnp.dot(p0, w0_ref[...], preferred_element_type=jnp.float32)
    a = a + jnp.dot(p1, w1_ref[...], preferred_element_type=jnp.float32)
    o_ref[...] = jnp.maximum(a + b_ref[...], 0.0)


def _tc_fuse(x, p, w_self, w_r0, w_r1, b, eps):
    R = 1000
    grid = N // R
    return pl.pallas_call(
        functools.partial(_fuse_body, eps),
        grid=(grid,),
        in_specs=[
            pl.BlockSpec((R, D), lambda i: (i, 0)),
            pl.BlockSpec((4, R, D), lambda i: (0, i, 0)),
            pl.BlockSpec((D, D), lambda i: (0, 0)),
            pl.BlockSpec((D, D), lambda i: (0, 0)),
            pl.BlockSpec((D, D), lambda i: (0, 0)),
            pl.BlockSpec((1, D), lambda i: (0, 0)),
        ],
        out_specs=pl.BlockSpec((R, D), lambda i: (i, 0)),
        out_shape=jax.ShapeDtypeStruct((N, D), jnp.float32),
    )(x, p, w_self, w_r0, w_r1, b)


def _bf16_packed(h):
    perm = (h.reshape(-1, 4, 2, 16).transpose(0, 1, 3, 2)
            .reshape(-1, D // 2, 2).astype(jnp.bfloat16))
    return lax.bitcast_convert_type(perm, jnp.int32)


def _pad_edges(ei):
    src = jnp.concatenate(
        [ei[0], jnp.zeros((E_PAD - E,), jnp.int32)]).reshape(E_PAD // B, B)
    dst = jnp.concatenate(
        [ei[1], jnp.full((E_PAD - E,), N, jnp.int32)]).reshape(E_PAD // B, B)
    return src, dst


def kernel(x, edge_index_rel0, edge_index_rel1, self_idx,
           W0_self, W0_r0, W0_r1, b0, W1_self, W1_r0, W1_r1, b1):
    del self_idx
    src0, dst0 = _pad_edges(edge_index_rel0)
    src1, dst1 = _pad_edges(edge_index_rel1)
    zeros = jnp.zeros((CH, D), jnp.float32)

    p = _sc_segsum(_bf16_packed(x), src0, dst0, src1, dst1,
                   zeros).reshape(4, N, D)
    h = _tc_fuse(x, p, W0_self, W0_r0, W0_r1, b0.reshape(1, D), EPS0)

    p = _sc_segsum(_bf16_packed(h), src0, dst0, src1, dst1,
                   zeros).reshape(4, N, D)
    out = _tc_fuse(h, p, W1_self, W1_r0, W1_r1, b1.reshape(1, D), EPS1)
    return out
